# Initial kernel scaffold; baseline (speedup 1.0000x reference)
import functools

import jax
import jax.numpy as jnp
from jax import lax
from jax.experimental import pallas as pl
from jax.experimental.pallas import tpu as pltpu
from jax.experimental.pallas import tpu_sc as plsc

N_FACES = 100000
N_EDGES = 150000
NNZ = 300000
D = 128

NSUB = 16
NCORE = 2
LANES = 16
K = 128
CHUNK = 8192
ROWS_PER_TILE = CHUNK // NSUB
NCH1 = -(-N_EDGES // CHUNK)
NCH2 = -(-N_FACES // CHUNK)
P1 = NCH1 * CHUNK
P2 = NCH2 * CHUNK
NNZP = NNZ + K
BIGDEST = 1 << 29

f32 = jnp.float32
i32 = jnp.int32


def _mm_pm_body(nblk, sigmoid_in, x_ref, w_ref, o_ref):
    xb = x_ref[...]
    if sigmoid_in:
        xb = jax.nn.sigmoid(xb)
    r = jnp.dot(xb, w_ref[...], precision=lax.Precision.HIGHEST,
                preferred_element_type=f32)
    o_ref[...] = jnp.where(pl.program_id(0) < nblk, r, -r)


def _mm_pm(x, w, bm, sigmoid_in):
    n = x.shape[0]
    nblk = n // bm
    return pl.pallas_call(
        functools.partial(_mm_pm_body, nblk, sigmoid_in),
        grid=(2 * nblk,),
        in_specs=[
            pl.BlockSpec((bm, D), lambda i: (lax.rem(i, nblk), 0)),
            pl.BlockSpec((D, D), lambda i: (0, 0)),
        ],
        out_specs=pl.BlockSpec((bm, D), lambda i: (i, 0)),
        out_shape=jax.ShapeDtypeStruct((2 * n, D), f32),
    )(x, w)


def _sigmoid_body(x_ref, o_ref):
    o_ref[...] = jax.nn.sigmoid(x_ref[...])


def _sigmoid_head(x, n, bm):
    return pl.pallas_call(
        _sigmoid_body,
        grid=(n // bm,),
        in_specs=[pl.BlockSpec((bm, D), lambda i: (i, 0))],
        out_specs=pl.BlockSpec((bm, D), lambda i: (i, 0)),
        out_shape=jax.ShapeDtypeStruct((n, D), f32),
    )(x)


def _vext(vec, c):
    sel = jnp.where(lax.iota(i32, LANES) == c, vec, -(1 << 30))
    return jnp.max(sel)


def _make_spmm(nch, out_rows):
    mesh = plsc.VectorSubcoreMesh(core_axis_name="c", subcore_axis_name="s")
    nloop = -(-nch // NCORE)

    @functools.partial(
        pl.kernel,
        mesh=mesh,
        out_type=jax.ShapeDtypeStruct((out_rows, D), f32),
        scratch_types=[
            pltpu.VMEM((32,), i32),
            pltpu.VMEM((K,), i32),
            pltpu.VMEM((K,), i32),
            pltpu.VMEM((K,), i32),
            pltpu.VMEM((K, D), f32),
            pltpu.VMEM((K, D), f32),
            pltpu.VMEM_SHARED((CHUNK + 8, D), f32),
        ],
    )
    def spmm(src_hbm, gidx_hbm, didx_hbm, bounds_hbm, zeros_hbm, out_hbm,
             bnd_v, gi_v, di_v, rel_v, gat_v, zero_v, acc):
        core = lax.axis_index("c")
        sub = lax.axis_index("s")

        pltpu.sync_copy(bounds_hbm, bnd_v)
        pltpu.sync_copy(zeros_hbm, zero_v)
        b0 = bnd_v[pl.ds(0, LANES)]
        b1 = bnd_v[pl.ds(LANES, LANES)]

        def bound(c):
            return jnp.where(c < LANES, _vext(b0, c), _vext(b1, c - LANES))

        def chunk_body(it, carry):
            c = core + NCORE * it

            @pl.when(c < nch)
            def _():
                base = c * CHUNK
                for q in range(ROWS_PER_TILE // K):
                    pltpu.sync_copy(
                        zero_v, acc.at[pl.ds(sub * ROWS_PER_TILE + q * K, K)])
                plsc.subcore_barrier()

                lo_c = bound(c) & ~7
                hi_c = (bound(c + 1) + 7) & ~7
                ln = hi_c - lo_c
                lo = lo_c + ((ln * sub) // NSUB & ~7)
                hi = lo_c + ((ln * (sub + 1)) // NSUB & ~7)
                nb = (hi - lo + K - 1) // K

                def batch(j, carry2):
                    pos = lo + j * K
                    pltpu.sync_copy(gidx_hbm.at[pl.ds(pos, K)], gi_v)
                    pltpu.sync_copy(didx_hbm.at[pl.ds(pos, K)], di_v)
                    for kk in range(K // LANES):
                        d = di_v[pl.ds(kk * LANES, LANES)]
                        gpos = pos + kk * LANES + lax.iota(i32, LANES)
                        rel = d - base
                        ok = (rel >= 0) & (rel < CHUNK) & (gpos < hi)
                        rel_v[pl.ds(kk * LANES, LANES)] = jnp.where(ok, rel, CHUNK)
                    pltpu.sync_copy(src_hbm.at[gi_v], gat_v)
                    pltpu.sync_copy(gat_v, acc.at[rel_v], add=True)
                    return carry2

                lax.fori_loop(0, nb, batch, 0)
                plsc.subcore_barrier()

                pltpu.sync_copy(
                    acc.at[pl.ds(sub * ROWS_PER_TILE, ROWS_PER_TILE)],
                    out_hbm.at[pl.ds(base + sub * ROWS_PER_TILE, ROWS_PER_TILE)])
            return carry

        lax.fori_loop(0, nloop, chunk_body, 0)

    return spmm


_spmm1 = _make_spmm(NCH1, P1)
_spmm2 = _make_spmm(NCH2, P2)


def _bucket(dest, gsrc, vals, nch, goffset):
    order = jnp.argsort(dest)
    d = dest[order].astype(i32)
    g = (gsrc[order] + goffset * (vals[order] < 0)).astype(i32)
    bounds = jnp.searchsorted(
        d, (jnp.arange(nch + 1, dtype=i32) * CHUNK)).astype(i32)
    d = jnp.concatenate([d, jnp.full((K,), BIGDEST, i32)])
    g = jnp.concatenate([g, jnp.zeros((K,), i32)])
    bounds = jnp.concatenate(
        [bounds, jnp.full((32 - nch - 1,), NNZ, i32)])
    return g, d, bounds


def kernel(x, rows, cols, vals, W1, W2):
    gidx1, didx1, bounds1 = _bucket(rows, cols, vals, NCH1, N_FACES)
    gidx2, didx2, bounds2 = _bucket(cols, rows, vals, NCH2, P1)
    zeros = jnp.zeros((K, D), f32)

    hpm = _mm_pm(x, W1, 1000, sigmoid_in=False)
    he = _spmm1(hpm, gidx1, didx1, bounds1, zeros)
    h2pm = _mm_pm(he, W2, 512, sigmoid_in=True)
    out = _spmm2(h2pm, gidx2, didx2, bounds2, zeros)
    return _sigmoid_head(out, N_FACES, 1000)


# speedup vs baseline: 1.7161x; 1.7161x over previous
import dataclasses
import functools

import jax
import jax.numpy as jnp
from jax import lax
from jax.experimental import pallas as pl
from jax.experimental.pallas import tpu as pltpu
from jax.experimental.pallas import tpu_sc as plsc

N_FACES = 100000
N_EDGES = 150000
NNZ = 300000
D = 128

NSUB = 16
NCORE = 2
LANES = 16
K = 128
CHUNK = 8192
ROWS_PER_TILE = CHUNK // NSUB
NCH1 = -(-N_EDGES // CHUNK)
NCH2 = -(-N_FACES // CHUNK)
P1 = NCH1 * CHUNK
P2 = NCH2 * CHUNK
NNZP = NNZ + K
BIGDEST = 1 << 29

f32 = jnp.float32
i32 = jnp.int32


def _mm_pm_body(nblk, sigmoid_in, x_ref, w_ref, o_ref):
    xb = x_ref[...]
    if sigmoid_in:
        xb = jax.nn.sigmoid(xb)
    r = jnp.dot(xb, w_ref[...], precision=lax.Precision.HIGHEST,
                preferred_element_type=f32)
    o_ref[...] = jnp.where(pl.program_id(0) < nblk, r, -r)


def _mm_pm(x, w, bm, sigmoid_in):
    n = x.shape[0]
    nblk = n // bm
    return pl.pallas_call(
        functools.partial(_mm_pm_body, nblk, sigmoid_in),
        grid=(2 * nblk,),
        in_specs=[
            pl.BlockSpec((bm, D), lambda i: (lax.rem(i, nblk), 0)),
            pl.BlockSpec((D, D), lambda i: (0, 0)),
        ],
        out_specs=pl.BlockSpec((bm, D), lambda i: (i, 0)),
        out_shape=jax.ShapeDtypeStruct((2 * n, D), f32),
    )(x, w)


def _sigmoid_body(x_ref, o_ref):
    o_ref[...] = jax.nn.sigmoid(x_ref[...])


def _sigmoid_head(x, n, bm):
    return pl.pallas_call(
        _sigmoid_body,
        grid=(n // bm,),
        in_specs=[pl.BlockSpec((bm, D), lambda i: (i, 0))],
        out_specs=pl.BlockSpec((bm, D), lambda i: (i, 0)),
        out_shape=jax.ShapeDtypeStruct((n, D), f32),
    )(x)


def _vext(vec, c):
    sel = jnp.where(lax.iota(i32, LANES) == c, vec, -(1 << 30))
    return jnp.max(sel)


def _make_spmm(nch, out_rows):
    mesh = plsc.VectorSubcoreMesh(core_axis_name="c", subcore_axis_name="s")
    nloop = -(-nch // NCORE)
    cp = pltpu.CompilerParams()
    if "needs_layout_passes" in pltpu.CompilerParams.__dataclass_fields__:
        cp = dataclasses.replace(cp, needs_layout_passes=False)

    @functools.partial(
        pl.kernel,
        mesh=mesh,
        compiler_params=cp,
        out_type=jax.ShapeDtypeStruct((out_rows, D), f32),
        scratch_types=[
            pltpu.VMEM((32,), i32),
            pltpu.VMEM((K,), i32),
            pltpu.VMEM((K,), i32),
            pltpu.VMEM((K,), i32),
            pltpu.VMEM((K, D), f32),
            pltpu.VMEM((K, D), f32),
            pltpu.VMEM_SHARED((CHUNK + 8, D), f32),
        ],
    )
    def spmm(src_hbm, gidx_hbm, didx_hbm, bounds_hbm, zeros_hbm, out_hbm,
             bnd_v, gi_v, di_v, rel_v, gat_v, zero_v, acc):
        core = lax.axis_index("c")
        sub = lax.axis_index("s")

        pltpu.sync_copy(bounds_hbm, bnd_v)
        pltpu.sync_copy(zeros_hbm, zero_v)
        b0 = bnd_v[pl.ds(0, LANES)]
        b1 = bnd_v[pl.ds(LANES, LANES)]

        def bound(c):
            return jnp.where(c < LANES, _vext(b0, c), _vext(b1, c - LANES))

        def chunk_body(it, carry):
            c = core + NCORE * it

            @pl.when(c < nch)
            def _():
                base = pl.multiple_of(c * CHUNK, CHUNK)
                for q in range(ROWS_PER_TILE // K):
                    pltpu.sync_copy(
                        zero_v,
                        acc.at[pl.ds(pl.multiple_of(
                            sub * ROWS_PER_TILE + q * K, K), K)])
                plsc.subcore_barrier()

                lo_c = pl.multiple_of(bound(c) & ~7, 8)
                hi_c = pl.multiple_of((bound(c + 1) + 7) & ~7, 8)
                ln = hi_c - lo_c
                lo = pl.multiple_of(lo_c + ((ln * sub) // NSUB & ~7), 8)
                hi = pl.multiple_of(lo_c + ((ln * (sub + 1)) // NSUB & ~7), 8)
                nb = (hi - lo + K - 1) // K

                def batch(j, carry2):
                    pos = pl.multiple_of(lo + j * K, 8)
                    pltpu.sync_copy(gidx_hbm.at[pl.ds(pos, K)], gi_v)
                    pltpu.sync_copy(didx_hbm.at[pl.ds(pos, K)], di_v)
                    for kk in range(K // LANES):
                        d = di_v[pl.ds(kk * LANES, LANES)]
                        gpos = pos + kk * LANES + lax.iota(i32, LANES)
                        rel = d - base
                        ok = (rel >= 0) & (rel < CHUNK) & (gpos < hi)
                        rel_v[pl.ds(kk * LANES, LANES)] = jnp.where(ok, rel, CHUNK)
                    pltpu.sync_copy(src_hbm.at[gi_v], gat_v)
                    pltpu.sync_copy(gat_v, acc.at[rel_v], add=True)
                    return carry2

                lax.fori_loop(0, nb, batch, 0)
                plsc.subcore_barrier()

                pltpu.sync_copy(
                    acc.at[pl.ds(pl.multiple_of(sub * ROWS_PER_TILE,
                                                ROWS_PER_TILE), ROWS_PER_TILE)],
                    out_hbm.at[pl.ds(pl.multiple_of(
                        base + sub * ROWS_PER_TILE, ROWS_PER_TILE),
                        ROWS_PER_TILE)])
            return carry

        lax.fori_loop(0, nloop, chunk_body, 0)

    return spmm


_spmm1 = _make_spmm(NCH1, P1)
_spmm2 = _make_spmm(NCH2, P2)


def _bucket(dest, gsrc, vals, nch, goffset):
    order = jnp.argsort(dest)
    d = dest[order].astype(i32)
    g = (gsrc[order] + goffset * (vals[order] < 0)).astype(i32)
    bounds = jnp.searchsorted(
        d, (jnp.arange(nch + 1, dtype=i32) * CHUNK)).astype(i32)
    d = jnp.concatenate([d, jnp.full((K,), BIGDEST, i32)])
    g = jnp.concatenate([g, jnp.zeros((K,), i32)])
    bounds = jnp.concatenate(
        [bounds, jnp.full((32 - nch - 1,), NNZ, i32)])
    return g, d, bounds


def kernel(x, rows, cols, vals, W1, W2):
    gidx1, didx1, bounds1 = _bucket(rows, cols, vals, NCH1, N_FACES)
    gidx2, didx2, bounds2 = _bucket(cols, rows, vals, NCH2, P1)
    zeros = jnp.zeros((K, D), f32)

    hpm = _mm_pm(x, W1, 1000, sigmoid_in=False)
    he = _spmm1(hpm, gidx1, didx1, bounds1, zeros)
    h2pm = _mm_pm(he, W2, 512, sigmoid_in=True)
    out = _spmm2(h2pm, gidx2, didx2, bounds2, zeros)
    return _sigmoid_head(out, N_FACES, 1000)
